# baseline (speedup 1.0000x reference)
import functools

import jax
import jax.numpy as jnp
from jax import lax
from jax.experimental import pallas as pl
from jax.experimental.pallas import tpu as pltpu
from jax.experimental.pallas import tpu_sc as plsc

E = 320000
H = 128
HG = H // 16
V = 6
NT = V * V * V
NC = 2
NS = 16
NW = NC * NS
BPW = E // NW
CHUNK = 80
NCHUNK = BPW // CHUNK
GRP = CHUNK // 16
NPAIR = (NCHUNK - 1) // 2


def _sc_body(cols_hbm, w0_hbm, w1_hbm, w2_hbm, out_hbm,
             a0_v, a1_v, a2_v, idx_v, w0_v, w1_v, w2_v, t12_v, tbl_v,
             r0, r1, sem, s1, s2, o0, o1):
    wid = lax.axis_index("s") * NC + lax.axis_index("c")
    ebase = wid * BPW

    pltpu.async_copy(cols_hbm.at[pl.ds(ebase, BPW)], a0_v, sem)
    pltpu.async_copy(cols_hbm.at[pl.ds(E + ebase, BPW)], a1_v, s1)
    pltpu.async_copy(cols_hbm.at[pl.ds(2 * E + ebase, BPW)], a2_v, s2)
    pltpu.sync_copy(w0_hbm, w0_v)
    pltpu.sync_copy(w1_hbm, w1_v)
    pltpu.sync_copy(w2_hbm, w2_v)

    def t12_row(r, _):
        j = r // V
        k = r % V
        for c in range(HG):
            t12_v[r, pl.ds(c * 16, 16)] = (
                w1_v[j, pl.ds(c * 16, 16)] + w2_v[k, pl.ds(c * 16, 16)]
            )
        return 0

    lax.fori_loop(0, V * V, t12_row, 0)

    def tbl_row(r, _):
        i = r // (V * V)
        m = r % (V * V)
        for c in range(HG):
            tbl_v[r, pl.ds(c * 16, 16)] = (
                w0_v[i, pl.ds(c * 16, 16)] + t12_v[m, pl.ds(c * 16, 16)]
            )
        return 0

    lax.fori_loop(0, NT, tbl_row, 0)

    pltpu.make_async_copy(cols_hbm.at[pl.ds(0, BPW)], a0_v, sem).wait()
    pltpu.make_async_copy(cols_hbm.at[pl.ds(0, BPW)], a1_v, s1).wait()
    pltpu.make_async_copy(cols_hbm.at[pl.ds(0, BPW)], a2_v, s2).wait()

    def idx_group(g, _):
        b = g * 16
        a0 = a0_v[pl.ds(b, 16)]
        a1 = a1_v[pl.ds(b, 16)]
        a2 = a2_v[pl.ds(b, 16)]
        idx_v[g // GRP, pl.ds((g % GRP) * 16, 16)] = a0 * 36 + a1 * 6 + a2
        return 0

    lax.fori_loop(0, NCHUNK * GRP, idx_group, 0)

    def expand(k, rv):
        def group(g, _):
            idx16 = idx_v[k, pl.ds(g * 16, 16)]
            e0 = g * 16
            for j in range(16):
                s = idx16[j]
                row = [tbl_v[s, pl.ds(c * 16, 16)] for c in range(HG)]
                for c in range(HG):
                    rv[e0 + j, pl.ds(c * 16, 16)] = row[c]
            return 0

        lax.fori_loop(0, GRP, group, 0)

    def out_start(k, rv, osem):
        return pltpu.async_copy(rv, out_hbm.at[pl.ds(ebase + k * CHUNK, CHUNK)], osem)

    def out_wait(rv, osem):
        pltpu.make_async_copy(rv, out_hbm.at[pl.ds(0, CHUNK)], osem).wait()

    expand(0, r0)
    out_start(0, r0, o0)

    def pair(i, _):
        a = 2 * i + 1
        b = 2 * i + 2

        @pl.when(i > 0)
        def _():
            out_wait(r1, o1)

        expand(a, r1)
        out_start(a, r1, o1)

        out_wait(r0, o0)
        expand(b, r0)
        out_start(b, r0, o0)
        return 0

    lax.fori_loop(0, NPAIR, pair, 0)
    out_wait(r1, o1)
    out_wait(r0, o0)


@functools.partial(jax.jit, donate_argnums=())
def kernel(edge_attr, W0, W1, W2):
    ea = edge_attr.astype(jnp.int32)
    cols = jnp.concatenate([ea[:, 0], ea[:, 1], ea[:, 2]])

    sc = pl.kernel(
        _sc_body,
        out_type=jax.ShapeDtypeStruct((E, H), jnp.float32),
        mesh=plsc.VectorSubcoreMesh(core_axis_name="c", subcore_axis_name="s"),
        compiler_params=pltpu.CompilerParams(needs_layout_passes=False),
        scratch_types=[
            pltpu.VMEM((BPW,), jnp.int32),
            pltpu.VMEM((BPW,), jnp.int32),
            pltpu.VMEM((BPW,), jnp.int32),
            pltpu.VMEM((NCHUNK, CHUNK), jnp.int32),
            pltpu.VMEM((V, H), jnp.float32),
            pltpu.VMEM((V, H), jnp.float32),
            pltpu.VMEM((V, H), jnp.float32),
            pltpu.VMEM((V * V, H), jnp.float32),
            pltpu.VMEM((NT, H), jnp.float32),
            pltpu.VMEM((CHUNK, H), jnp.float32),
            pltpu.VMEM((CHUNK, H), jnp.float32),
            pltpu.SemaphoreType.DMA,
            pltpu.SemaphoreType.DMA,
            pltpu.SemaphoreType.DMA,
            pltpu.SemaphoreType.DMA,
            pltpu.SemaphoreType.DMA,
        ],
    )
    return sc(cols, W0, W1, W2)


# speedup vs baseline: 1.0021x; 1.0021x over previous
import functools

import jax
import jax.numpy as jnp
from jax import lax
from jax.experimental import pallas as pl
from jax.experimental.pallas import tpu as pltpu
from jax.experimental.pallas import tpu_sc as plsc

E = 320000
H = 128
HG = H // 16
V = 6
NT = V * V * V
NC = 2
NS = 16
NW = NC * NS
BPW = E // NW
CHUNK = 80
NCHUNK = BPW // CHUNK
GRP = CHUNK // 16
NPAIR = (NCHUNK - 1) // 2


def _sc_body(a0_hbm, a1_hbm, a2_hbm, w0_hbm, w1_hbm, w2_hbm, out_hbm,
             a0_v, a1_v, a2_v, idx_v, w0_v, w1_v, w2_v, t12_v, tbl_v,
             r0, r1, sem, s1, s2, o0, o1):
    wid = lax.axis_index("s") * NC + lax.axis_index("c")
    ebase = wid * BPW

    pltpu.async_copy(a0_hbm.at[pl.ds(ebase, BPW)], a0_v, sem)
    pltpu.async_copy(a1_hbm.at[pl.ds(ebase, BPW)], a1_v, s1)
    pltpu.async_copy(a2_hbm.at[pl.ds(ebase, BPW)], a2_v, s2)
    pltpu.sync_copy(w0_hbm, w0_v)
    pltpu.sync_copy(w1_hbm, w1_v)
    pltpu.sync_copy(w2_hbm, w2_v)

    def t12_row(r, _):
        j = r // V
        k = r % V
        for c in range(HG):
            t12_v[r, pl.ds(c * 16, 16)] = (
                w1_v[j, pl.ds(c * 16, 16)] + w2_v[k, pl.ds(c * 16, 16)]
            )
        return 0

    lax.fori_loop(0, V * V, t12_row, 0)

    def tbl_row(r, _):
        i = r // (V * V)
        m = r % (V * V)
        for c in range(HG):
            tbl_v[r, pl.ds(c * 16, 16)] = (
                w0_v[i, pl.ds(c * 16, 16)] + t12_v[m, pl.ds(c * 16, 16)]
            )
        return 0

    lax.fori_loop(0, NT, tbl_row, 0)

    pltpu.make_async_copy(a0_hbm.at[pl.ds(0, BPW)], a0_v, sem).wait()
    pltpu.make_async_copy(a1_hbm.at[pl.ds(0, BPW)], a1_v, s1).wait()
    pltpu.make_async_copy(a2_hbm.at[pl.ds(0, BPW)], a2_v, s2).wait()

    def idx_group(g, _):
        b = g * 16
        a0 = a0_v[pl.ds(b, 16)]
        a1 = a1_v[pl.ds(b, 16)]
        a2 = a2_v[pl.ds(b, 16)]
        idx_v[g // GRP, pl.ds((g % GRP) * 16, 16)] = a0 * 36 + a1 * 6 + a2
        return 0

    lax.fori_loop(0, NCHUNK * GRP, idx_group, 0)

    def expand(k, rv):
        def group(g, _):
            idx16 = idx_v[k, pl.ds(g * 16, 16)]
            e0 = g * 16
            for j in range(16):
                s = idx16[j]
                row = [tbl_v[s, pl.ds(c * 16, 16)] for c in range(HG)]
                for c in range(HG):
                    rv[e0 + j, pl.ds(c * 16, 16)] = row[c]
            return 0

        lax.fori_loop(0, GRP, group, 0)

    def out_start(k, rv, osem):
        return pltpu.async_copy(rv, out_hbm.at[pl.ds(ebase + k * CHUNK, CHUNK)], osem)

    def out_wait(rv, osem):
        pltpu.make_async_copy(rv, out_hbm.at[pl.ds(0, CHUNK)], osem).wait()

    expand(0, r0)
    out_start(0, r0, o0)

    def pair(i, _):
        a = 2 * i + 1
        b = 2 * i + 2

        @pl.when(i > 0)
        def _():
            out_wait(r1, o1)

        expand(a, r1)
        out_start(a, r1, o1)

        out_wait(r0, o0)
        expand(b, r0)
        out_start(b, r0, o0)
        return 0

    lax.fori_loop(0, NPAIR, pair, 0)
    out_wait(r1, o1)
    out_wait(r0, o0)


@functools.partial(jax.jit, donate_argnums=())
def kernel(edge_attr, W0, W1, W2):
    ea = edge_attr.astype(jnp.int32)
    a0 = ea[:, 0]
    a1 = ea[:, 1]
    a2 = ea[:, 2]

    sc = pl.kernel(
        _sc_body,
        out_type=jax.ShapeDtypeStruct((E, H), jnp.float32),
        mesh=plsc.VectorSubcoreMesh(core_axis_name="c", subcore_axis_name="s"),
        compiler_params=pltpu.CompilerParams(needs_layout_passes=False),
        scratch_types=[
            pltpu.VMEM((BPW,), jnp.int32),
            pltpu.VMEM((BPW,), jnp.int32),
            pltpu.VMEM((BPW,), jnp.int32),
            pltpu.VMEM((NCHUNK, CHUNK), jnp.int32),
            pltpu.VMEM((V, H), jnp.float32),
            pltpu.VMEM((V, H), jnp.float32),
            pltpu.VMEM((V, H), jnp.float32),
            pltpu.VMEM((V * V, H), jnp.float32),
            pltpu.VMEM((NT, H), jnp.float32),
            pltpu.VMEM((CHUNK, H), jnp.float32),
            pltpu.VMEM((CHUNK, H), jnp.float32),
            pltpu.SemaphoreType.DMA,
            pltpu.SemaphoreType.DMA,
            pltpu.SemaphoreType.DMA,
            pltpu.SemaphoreType.DMA,
            pltpu.SemaphoreType.DMA,
        ],
    )
    return sc(a0, a1, a2, W0, W1, W2)


# speedup vs baseline: 1.0179x; 1.0157x over previous
import functools

import jax
import jax.numpy as jnp
from jax import lax
from jax.experimental import pallas as pl
from jax.experimental.pallas import tpu as pltpu
from jax.experimental.pallas import tpu_sc as plsc

E = 320000
H = 128
HG = H // 16
V = 6
NT = V * V * V
NC = 2
NS = 16
NW = NC * NS
BPW = E // NW
CHUNK = 80
NCHUNK = BPW // CHUNK
GRP = CHUNK // 16
NPAIR = (NCHUNK - 1) // 2


def _sc_body(a0_hbm, a1_hbm, a2_hbm, w0_hbm, w1_hbm, w2_hbm, out_hbm,
             a0_v, a1_v, a2_v, w0_v, w1_v, w2_v, t12_v, tbl_v,
             r0, r1, sem, s1, s2, o0, o1):
    wid = lax.axis_index("s") * NC + lax.axis_index("c")
    ebase = wid * BPW

    pltpu.async_copy(a0_hbm.at[pl.ds(ebase, BPW)], a0_v, sem)
    pltpu.async_copy(a1_hbm.at[pl.ds(ebase, BPW)], a1_v, s1)
    pltpu.async_copy(a2_hbm.at[pl.ds(ebase, BPW)], a2_v, s2)
    pltpu.sync_copy(w0_hbm, w0_v)
    pltpu.sync_copy(w1_hbm, w1_v)
    pltpu.sync_copy(w2_hbm, w2_v)

    def t12_row(r, _):
        j = r // V
        k = r % V
        for c in range(HG):
            t12_v[r, pl.ds(c * 16, 16)] = (
                w1_v[j, pl.ds(c * 16, 16)] + w2_v[k, pl.ds(c * 16, 16)]
            )
        return 0

    lax.fori_loop(0, V * V, t12_row, 0)

    def tbl_row(r, _):
        i = r // (V * V)
        m = r % (V * V)
        for c in range(HG):
            tbl_v[r, pl.ds(c * 16, 16)] = (
                w0_v[i, pl.ds(c * 16, 16)] + t12_v[m, pl.ds(c * 16, 16)]
            )
        return 0

    lax.fori_loop(0, NT, tbl_row, 0)

    pltpu.make_async_copy(a0_hbm.at[pl.ds(0, BPW)], a0_v, sem).wait()
    pltpu.make_async_copy(a1_hbm.at[pl.ds(0, BPW)], a1_v, s1).wait()
    pltpu.make_async_copy(a2_hbm.at[pl.ds(0, BPW)], a2_v, s2).wait()

    def expand(k, rv):
        def group(g, _):
            b = k * CHUNK + g * 16
            idx16 = (
                a0_v[pl.ds(b, 16)] * 36
                + a1_v[pl.ds(b, 16)] * 6
                + a2_v[pl.ds(b, 16)]
            )
            e0 = g * 16
            for j in range(16):
                s = idx16[j]
                row = [tbl_v[s, pl.ds(c * 16, 16)] for c in range(HG)]
                for c in range(HG):
                    rv[e0 + j, pl.ds(c * 16, 16)] = row[c]
            return 0

        lax.fori_loop(0, GRP, group, 0)

    def out_start(k, rv, osem):
        return pltpu.async_copy(rv, out_hbm.at[pl.ds(ebase + k * CHUNK, CHUNK)], osem)

    def out_wait(rv, osem):
        pltpu.make_async_copy(rv, out_hbm.at[pl.ds(0, CHUNK)], osem).wait()

    expand(0, r0)
    out_start(0, r0, o0)

    def pair(i, _):
        a = 2 * i + 1
        b = 2 * i + 2

        @pl.when(i > 0)
        def _():
            out_wait(r1, o1)

        expand(a, r1)
        out_start(a, r1, o1)

        out_wait(r0, o0)
        expand(b, r0)
        out_start(b, r0, o0)
        return 0

    lax.fori_loop(0, NPAIR, pair, 0)
    out_wait(r1, o1)
    out_wait(r0, o0)


@functools.partial(jax.jit, donate_argnums=())
def kernel(edge_attr, W0, W1, W2):
    ea = edge_attr.astype(jnp.int32)
    a0 = ea[:, 0]
    a1 = ea[:, 1]
    a2 = ea[:, 2]

    sc = pl.kernel(
        _sc_body,
        out_type=jax.ShapeDtypeStruct((E, H), jnp.float32),
        mesh=plsc.VectorSubcoreMesh(core_axis_name="c", subcore_axis_name="s"),
        compiler_params=pltpu.CompilerParams(needs_layout_passes=False),
        scratch_types=[
            pltpu.VMEM((BPW,), jnp.int32),
            pltpu.VMEM((BPW,), jnp.int32),
            pltpu.VMEM((BPW,), jnp.int32),
            pltpu.VMEM((V, H), jnp.float32),
            pltpu.VMEM((V, H), jnp.float32),
            pltpu.VMEM((V, H), jnp.float32),
            pltpu.VMEM((V * V, H), jnp.float32),
            pltpu.VMEM((NT, H), jnp.float32),
            pltpu.VMEM((CHUNK, H), jnp.float32),
            pltpu.VMEM((CHUNK, H), jnp.float32),
            pltpu.SemaphoreType.DMA,
            pltpu.SemaphoreType.DMA,
            pltpu.SemaphoreType.DMA,
            pltpu.SemaphoreType.DMA,
            pltpu.SemaphoreType.DMA,
        ],
    )
    return sc(a0, a1, a2, W0, W1, W2)
